# baseline (speedup 1.0000x reference)
import jax
import jax.numpy as jnp
from jax.experimental import pallas as pl
from jax.experimental.pallas import tpu as pltpu

NUM_CLASSES = 1000
ROWS_PER_BLOCK = 32


def _onehot_body(x_ref, out_ref):
    x = x_ref[...]
    classes = jax.lax.broadcasted_iota(
        jnp.int32, (ROWS_PER_BLOCK, 26, NUM_CLASSES), 2
    )
    out_ref[...] = (x[:, :, None] == classes).astype(jnp.float32)


def kernel(x):
    B, S = x.shape
    x = x.astype(jnp.int32)
    grid = (B // ROWS_PER_BLOCK,)
    return pl.pallas_call(
        _onehot_body,
        grid=grid,
        in_specs=[pl.BlockSpec((ROWS_PER_BLOCK, S), lambda i: (i, 0))],
        out_specs=pl.BlockSpec((ROWS_PER_BLOCK, S, NUM_CLASSES), lambda i: (i, 0, 0)),
        out_shape=jax.ShapeDtypeStruct((B, S, NUM_CLASSES), jnp.float32),
    )(x)
